# Initial kernel scaffold; baseline (speedup 1.0000x reference)
import jax
import jax.numpy as jnp
from jax.experimental import pallas as pl


def kernel(h, edge_index, norm, weight, bias):
    raise NotImplementedError("write your pallas kernel here")


# speedup vs baseline: 5.2813x; 5.2813x over previous
import functools

import jax
import jax.numpy as jnp
from jax import lax
from jax.experimental import pallas as pl
from jax.experimental.pallas import tpu as pltpu
from jax.experimental.pallas import tpu_sc as plsc

N_NODES = 10000
N_EDGES = 320000
F = 128
NC = 2
NS = 16
EC = 80
EPT = N_EDGES // (NC * NS)
NCHUNK = EPT // EC
NP = 10240
ROWS_PT = NP // NS
BR = 2000
BRF = 640


def _tc_matmul_body(h_ref, w_ref, norm_ref, out_ref):
    out_ref[...] = jnp.dot(h_ref[...], w_ref[...],
                           preferred_element_type=jnp.float32) * norm_ref[...]


def _tc_matmul(h, weight, norm):
    grid = (N_NODES // BR,)
    return pl.pallas_call(
        _tc_matmul_body,
        grid=grid,
        in_specs=[
            pl.BlockSpec((BR, F), lambda r: (r, 0)),
            pl.BlockSpec((F, F), lambda r: (0, 0)),
            pl.BlockSpec((BR, 1), lambda r: (r, 0)),
        ],
        out_specs=pl.BlockSpec((BR, F), lambda r: (r, 0)),
        out_shape=jax.ShapeDtypeStruct((N_NODES, F), jnp.float32),
    )(h, weight, norm)


def _tc_final_body(a0_ref, a1_ref, norm_ref, bias_ref, out_ref):
    agg = a0_ref[...] + a1_ref[...]
    out_ref[...] = agg * norm_ref[...] + bias_ref[...]


def _tc_final(parts, norm, bias2):
    grid = (NP // BRF,)
    nb = NP // BRF
    return pl.pallas_call(
        _tc_final_body,
        grid=grid,
        in_specs=[
            pl.BlockSpec((BRF, F), lambda r: (r, 0)),
            pl.BlockSpec((BRF, F), lambda r: (nb + r, 0)),
            pl.BlockSpec((BRF, 1), lambda r: (r, 0)),
            pl.BlockSpec((1, F), lambda r: (0, 0)),
        ],
        out_specs=pl.BlockSpec((BRF, F), lambda r: (r, 0)),
        out_shape=jax.ShapeDtypeStruct((N_NODES, F), jnp.float32),
    )(parts, parts, norm, bias2)


def _sc_agg_body(table_hbm, src_hbm, dst_hbm, zeros_hbm, out_hbm,
                 src_v, dst_v, rows_v, acc_sh, sem):
    c = lax.axis_index("c")
    s = lax.axis_index("s")

    pltpu.sync_copy(zeros_hbm.at[pl.ds(s * ROWS_PT, ROWS_PT)],
                    acc_sh.at[pl.ds(s * ROWS_PT, ROWS_PT)])
    plsc.subcore_barrier()

    base_e = (c * NS + s) * EPT

    def chunk(j, carry):
        off = base_e + j * EC
        pltpu.sync_copy(src_hbm.at[pl.ds(off, EC)], src_v)
        pltpu.sync_copy(dst_hbm.at[pl.ds(off, EC)], dst_v)
        pltpu.async_copy(table_hbm.at[src_v], rows_v, sem).wait()
        pltpu.sync_copy(rows_v, acc_sh.at[dst_v], add=True)
        return carry

    lax.fori_loop(0, NCHUNK, chunk, 0)
    plsc.subcore_barrier()

    pltpu.sync_copy(acc_sh.at[pl.ds(s * ROWS_PT, ROWS_PT)],
                    out_hbm.at[pl.ds(c * NP + s * ROWS_PT, ROWS_PT)])


_sc_agg = functools.partial(
    pl.kernel,
    mesh=plsc.VectorSubcoreMesh(core_axis_name="c", subcore_axis_name="s",
                                num_cores=NC, num_subcores=NS),
    out_type=jax.ShapeDtypeStruct((NC * NP, F), jnp.float32),
    scratch_types=[
        pltpu.VMEM((EC,), jnp.int32),
        pltpu.VMEM((EC,), jnp.int32),
        pltpu.VMEM((EC, F), jnp.float32),
        pltpu.VMEM_SHARED((NP, F), jnp.float32),
        pltpu.SemaphoreType.DMA,
    ],
)(_sc_agg_body)


def kernel(h, edge_index, norm, weight, bias):
    src = edge_index[0].astype(jnp.int32)
    dst = edge_index[1].astype(jnp.int32)
    hw = _tc_matmul(h, weight, norm)
    zeros = jnp.zeros((NP, F), jnp.float32)
    parts = _sc_agg(hw, src, dst, zeros)
    return _tc_final(parts, norm, bias.reshape(1, F))


# speedup vs baseline: 10.8207x; 2.0489x over previous
import functools

import jax
import jax.numpy as jnp
from jax import lax
from jax.experimental import pallas as pl
from jax.experimental.pallas import tpu as pltpu
from jax.experimental.pallas import tpu_sc as plsc

N_NODES = 10000
N_EDGES = 320000
F = 128
NC = 2
NS = 16
EC = 128
CPS = 8
SCT = 10
CHT = CPS * SCT
E_PAD = NC * NS * CHT * EC
NP = 10240
ROWS_PT = NP // NS
BR = 2000
BRF = 640


def _tc_matmul_body(h_ref, w_ref, norm_ref, out_ref):
    out_ref[...] = jnp.dot(h_ref[...], w_ref[...],
                           preferred_element_type=jnp.float32) * norm_ref[...]


def _tc_matmul(h, weight, norm):
    grid = (N_NODES // BR,)
    return pl.pallas_call(
        _tc_matmul_body,
        grid=grid,
        in_specs=[
            pl.BlockSpec((BR, F), lambda r: (r, 0)),
            pl.BlockSpec((F, F), lambda r: (0, 0)),
            pl.BlockSpec((BR, 1), lambda r: (r, 0)),
        ],
        out_specs=pl.BlockSpec((BR, F), lambda r: (r, 0)),
        out_shape=jax.ShapeDtypeStruct((N_NODES, F), jnp.float32),
    )(h, weight, norm)


def _tc_final_body(a0_ref, a1_ref, norm_ref, bias_ref, out_ref):
    agg = a0_ref[...] + a1_ref[...]
    out_ref[...] = agg * norm_ref[...] + bias_ref[...]


def _tc_final(parts, norm, bias2):
    grid = (NP // BRF,)
    nb = NP // BRF
    return pl.pallas_call(
        _tc_final_body,
        grid=grid,
        in_specs=[
            pl.BlockSpec((BRF, F), lambda r: (r, 0)),
            pl.BlockSpec((BRF, F), lambda r: (nb + r, 0)),
            pl.BlockSpec((BRF, 1), lambda r: (r, 0)),
            pl.BlockSpec((1, F), lambda r: (0, 0)),
        ],
        out_specs=pl.BlockSpec((BRF, F), lambda r: (r, 0)),
        out_shape=jax.ShapeDtypeStruct((N_NODES, F), jnp.float32),
    )(parts, parts, norm, bias2)


def _sc_agg_body(table_hbm, src_hbm, dst_hbm, zeros_hbm, out_hbm,
                 ibs0, ibs1, ibd0, ibd1, rows0, rows1, acc_sh,
                 isem0, isem1, gs0, gs1, cs0, cs1):
    c = lax.axis_index("c")
    s = lax.axis_index("s")
    wid = c * NS + s
    base = wid * CHT
    ibs = (ibs0, ibs1)
    ibd = (ibd0, ibd1)
    isem = (isem0, isem1)
    rows = (rows0, rows1)
    gsem = (gs0, gs1)
    csem = (cs0, cs1)

    zd = pltpu.async_copy(zeros_hbm.at[pl.ds(s * ROWS_PT, ROWS_PT)],
                          acc_sh.at[pl.ds(s * ROWS_PT, ROWS_PT)], gs0)
    for par in range(2):
        pltpu.async_copy(src_hbm.at[pl.ds(base + par * CPS, CPS)],
                         ibs[par], isem[par])
        pltpu.async_copy(dst_hbm.at[pl.ds(base + par * CPS, CPS)],
                         ibd[par], isem[par])
    zd.wait()
    plsc.subcore_barrier()

    def two_supers(t, carry):
        for par in range(2):
            sc_i = 2 * t + par
            pltpu.make_async_copy(src_hbm.at[pl.ds(base, CPS)],
                                  ibs[par], isem[par]).wait()
            pltpu.make_async_copy(dst_hbm.at[pl.ds(base, CPS)],
                                  ibd[par], isem[par]).wait()
            gd = {}
            cd = {}
            for b in range(2):
                gd[b] = pltpu.async_copy(table_hbm.at[ibs[par].at[b]],
                                         rows[b], gsem[b])
            for b in range(CPS):
                gd[b].wait()
                cd[b] = pltpu.async_copy(rows[b % 2],
                                         acc_sh.at[ibd[par].at[b]],
                                         csem[b % 2], add=True)
                if b + 2 < CPS:
                    cd[b].wait()
                    gd[b + 2] = pltpu.async_copy(
                        table_hbm.at[ibs[par].at[b + 2]],
                        rows[b % 2], gsem[b % 2])
            cd[CPS - 2].wait()
            cd[CPS - 1].wait()
            nxt = base + jnp.minimum(sc_i + 2, SCT - 1) * CPS
            pltpu.async_copy(src_hbm.at[pl.ds(nxt, CPS)], ibs[par], isem[par])
            pltpu.async_copy(dst_hbm.at[pl.ds(nxt, CPS)], ibd[par], isem[par])
        return carry

    lax.fori_loop(0, SCT // 2, two_supers, 0)
    for par in range(2):
        pltpu.make_async_copy(src_hbm.at[pl.ds(base, CPS)],
                              ibs[par], isem[par]).wait()
        pltpu.make_async_copy(dst_hbm.at[pl.ds(base, CPS)],
                              ibd[par], isem[par]).wait()
    plsc.subcore_barrier()

    pltpu.sync_copy(acc_sh.at[pl.ds(s * ROWS_PT, ROWS_PT)],
                    out_hbm.at[pl.ds(c * NP + s * ROWS_PT, ROWS_PT)])


_sc_agg = functools.partial(
    pl.kernel,
    mesh=plsc.VectorSubcoreMesh(core_axis_name="c", subcore_axis_name="s",
                                num_cores=NC, num_subcores=NS),
    out_type=jax.ShapeDtypeStruct((NC * NP, F), jnp.float32),
    scratch_types=[
        pltpu.VMEM((CPS, EC), jnp.int32),
        pltpu.VMEM((CPS, EC), jnp.int32),
        pltpu.VMEM((CPS, EC), jnp.int32),
        pltpu.VMEM((CPS, EC), jnp.int32),
        pltpu.VMEM((EC, F), jnp.float32),
        pltpu.VMEM((EC, F), jnp.float32),
        pltpu.VMEM_SHARED((NP, F), jnp.float32),
    ] + [pltpu.SemaphoreType.DMA for _ in range(6)],
)(_sc_agg_body)


def kernel(h, edge_index, norm, weight, bias):
    src = edge_index[0].astype(jnp.int32)
    dst = edge_index[1].astype(jnp.int32)
    npad = E_PAD - N_EDGES
    pad_iota = lax.iota(jnp.int32, npad)
    src_p = jnp.concatenate([src, pad_iota % N_NODES]).reshape(-1, EC)
    dst_p = jnp.concatenate([dst, N_NODES + pad_iota % (NP - N_NODES)]
                            ).reshape(-1, EC)
    hw = _tc_matmul(h, weight, norm)
    zeros = jnp.zeros((NP, F), jnp.float32)
    parts = _sc_agg(hw, src_p, dst_p, zeros)
    return _tc_final(parts, norm, bias.reshape(1, F))


# speedup vs baseline: 11.3512x; 1.0490x over previous
import functools

import jax
import jax.numpy as jnp
from jax import lax
from jax.experimental import pallas as pl
from jax.experimental.pallas import tpu as pltpu
from jax.experimental.pallas import tpu_sc as plsc

N_NODES = 10000
N_EDGES = 320000
F = 128
NC = 2
NS = 16
EC = 64
CPS = 16
NSLOT = 4
SCT = 10
CHT = CPS * SCT
E_PAD = NC * NS * CHT * EC
NP = 10240
ROWS_PT = NP // NS
BR = 2000
BRF = 640


def _tc_matmul_body(h_ref, w_ref, norm_ref, out_ref):
    out_ref[...] = jnp.dot(h_ref[...], w_ref[...],
                           preferred_element_type=jnp.float32) * norm_ref[...]


def _tc_matmul(h, weight, norm):
    grid = (N_NODES // BR,)
    return pl.pallas_call(
        _tc_matmul_body,
        grid=grid,
        in_specs=[
            pl.BlockSpec((BR, F), lambda r: (r, 0)),
            pl.BlockSpec((F, F), lambda r: (0, 0)),
            pl.BlockSpec((BR, 1), lambda r: (r, 0)),
        ],
        out_specs=pl.BlockSpec((BR, F), lambda r: (r, 0)),
        out_shape=jax.ShapeDtypeStruct((N_NODES, F), jnp.float32),
    )(h, weight, norm)


def _tc_final_body(a0_ref, a1_ref, norm_ref, bias_ref, out_ref):
    agg = a0_ref[...] + a1_ref[...]
    out_ref[...] = agg * norm_ref[...] + bias_ref[...]


def _tc_final(parts, norm, bias2):
    grid = (NP // BRF,)
    nb = NP // BRF
    return pl.pallas_call(
        _tc_final_body,
        grid=grid,
        in_specs=[
            pl.BlockSpec((BRF, F), lambda r: (r, 0)),
            pl.BlockSpec((BRF, F), lambda r: (nb + r, 0)),
            pl.BlockSpec((BRF, 1), lambda r: (r, 0)),
            pl.BlockSpec((1, F), lambda r: (0, 0)),
        ],
        out_specs=pl.BlockSpec((BRF, F), lambda r: (r, 0)),
        out_shape=jax.ShapeDtypeStruct((N_NODES, F), jnp.float32),
    )(parts, parts, norm, bias2)


def _sc_agg_body(table_hbm, src_hbm, dst_hbm, zeros_hbm, out_hbm,
                 ibs0, ibs1, ibd0, ibd1, rows0, rows1, rows2, rows3, acc_sh,
                 isem0, isem1, gs0, gs1, gs2, gs3, cs0, cs1, cs2, cs3):
    c = lax.axis_index("c")
    s = lax.axis_index("s")
    wid = c * NS + s
    base = wid * CHT
    ibs = (ibs0, ibs1)
    ibd = (ibd0, ibd1)
    isem = (isem0, isem1)
    rows = (rows0, rows1, rows2, rows3)
    gsem = (gs0, gs1, gs2, gs3)
    csem = (cs0, cs1, cs2, cs3)

    zd = pltpu.async_copy(zeros_hbm.at[pl.ds(s * ROWS_PT, ROWS_PT)],
                          acc_sh.at[pl.ds(s * ROWS_PT, ROWS_PT)], gs0)
    for par in range(2):
        pltpu.async_copy(src_hbm.at[pl.ds(base + par * CPS, CPS)],
                         ibs[par], isem[par])
        pltpu.async_copy(dst_hbm.at[pl.ds(base + par * CPS, CPS)],
                         ibd[par], isem[par])
    zd.wait()
    plsc.subcore_barrier()

    def two_supers(t, carry):
        for par in range(2):
            sc_i = 2 * t + par
            pltpu.make_async_copy(src_hbm.at[pl.ds(base, CPS)],
                                  ibs[par], isem[par]).wait()
            pltpu.make_async_copy(dst_hbm.at[pl.ds(base, CPS)],
                                  ibd[par], isem[par]).wait()
            gd = {}
            cd = {}
            for b in range(NSLOT):
                gd[b] = pltpu.async_copy(table_hbm.at[ibs[par].at[b]],
                                         rows[b], gsem[b])
            for b in range(CPS):
                gd[b].wait()
                cd[b] = pltpu.async_copy(rows[b % NSLOT],
                                         acc_sh.at[ibd[par].at[b]],
                                         csem[b % NSLOT], add=True)
                if b + NSLOT < CPS:
                    cd[b].wait()
                    gd[b + NSLOT] = pltpu.async_copy(
                        table_hbm.at[ibs[par].at[b + NSLOT]],
                        rows[b % NSLOT], gsem[b % NSLOT])
            for b in range(CPS - NSLOT, CPS):
                cd[b].wait()
            nxt = base + jnp.minimum(sc_i + 2, SCT - 1) * CPS
            pltpu.async_copy(src_hbm.at[pl.ds(nxt, CPS)], ibs[par], isem[par])
            pltpu.async_copy(dst_hbm.at[pl.ds(nxt, CPS)], ibd[par], isem[par])
        return carry

    lax.fori_loop(0, SCT // 2, two_supers, 0)
    for par in range(2):
        pltpu.make_async_copy(src_hbm.at[pl.ds(base, CPS)],
                              ibs[par], isem[par]).wait()
        pltpu.make_async_copy(dst_hbm.at[pl.ds(base, CPS)],
                              ibd[par], isem[par]).wait()
    plsc.subcore_barrier()

    pltpu.sync_copy(acc_sh.at[pl.ds(s * ROWS_PT, ROWS_PT)],
                    out_hbm.at[pl.ds(c * NP + s * ROWS_PT, ROWS_PT)])


_sc_agg = functools.partial(
    pl.kernel,
    mesh=plsc.VectorSubcoreMesh(core_axis_name="c", subcore_axis_name="s",
                                num_cores=NC, num_subcores=NS),
    out_type=jax.ShapeDtypeStruct((NC * NP, F), jnp.float32),
    scratch_types=[
        pltpu.VMEM((CPS, EC), jnp.int32),
        pltpu.VMEM((CPS, EC), jnp.int32),
        pltpu.VMEM((CPS, EC), jnp.int32),
        pltpu.VMEM((CPS, EC), jnp.int32),
    ] + [pltpu.VMEM((EC, F), jnp.float32) for _ in range(NSLOT)] + [
        pltpu.VMEM_SHARED((NP, F), jnp.float32),
    ] + [pltpu.SemaphoreType.DMA for _ in range(2 + 2 * NSLOT)],
)(_sc_agg_body)


def kernel(h, edge_index, norm, weight, bias):
    src = edge_index[0].astype(jnp.int32)
    dst = edge_index[1].astype(jnp.int32)
    npad = E_PAD - N_EDGES
    pad_iota = lax.iota(jnp.int32, npad)
    src_p = jnp.concatenate([src, pad_iota % N_NODES]).reshape(-1, EC)
    dst_p = jnp.concatenate([dst, N_NODES + pad_iota % (NP - N_NODES)]
                            ).reshape(-1, EC)
    hw = _tc_matmul(h, weight, norm)
    zeros = jnp.zeros((NP, F), jnp.float32)
    parts = _sc_agg(hw, src_p, dst_p, zeros)
    return _tc_final(parts, norm, bias.reshape(1, F))
